# baseline (speedup 1.0000x reference)
import functools

import jax
import jax.numpy as jnp
from jax import lax
from jax.experimental import pallas as pl
from jax.experimental.pallas import tpu as pltpu
from jax.experimental.pallas import tpu_sc as plsc

_N = 10000
_E = 320000
_EMBED = 128
_HID = 256
_NG = 50
_RBF_R = 12.0
_AVG_LEN = 60.0
_CONN = 32.0
_NGRAPH = 64

_NC = 2
_NS = 16
_L = 16
_NW = _NC * _NS
_EPW = _E // _NW

_GCH = 80
_GITER = _EPW // _GCH

_SCH = 400
_SITER = _EPW // _SCH

_NP = 10240
_ACC = 4 * _NP

_BE = 512


_XW = _EMBED // 2


def _gather_body(x_hbm, src_hbm, dst_hbm, xs_out, xd_out,
                 idx_s, idx_d, rows_s, rows_d, sem_s, sem_d):
    wid = lax.axis_index("s") * _NC + lax.axis_index("c")

    def step(c, carry):
        base = wid * _EPW + c * _GCH
        pltpu.sync_copy(src_hbm.at[pl.ds(base, _GCH)], idx_s)
        pltpu.sync_copy(dst_hbm.at[pl.ds(base, _GCH)], idx_d)
        cp_s = pltpu.async_copy(x_hbm.at[idx_s], rows_s, sem_s)
        cp_d = pltpu.async_copy(x_hbm.at[idx_d], rows_d, sem_d)
        cp_s.wait()
        pltpu.sync_copy(rows_s, xs_out.at[pl.ds(base, _GCH)])
        cp_d.wait()
        pltpu.sync_copy(rows_d, xd_out.at[pl.ds(base, _GCH)])
        return carry

    lax.fori_loop(0, _GITER, step, 0)


def _scatter_body(pairs_hbm, src_hbm, part_out, acc, pairs_v, src_v):
    wid = lax.axis_index("s") * _NC + lax.axis_index("c")
    zero = jnp.zeros((_L,), jnp.float32)

    def zstep(i, carry):
        acc[pl.ds(i * _L, _L)] = zero
        return carry

    lax.fori_loop(0, _ACC // _L, zstep, 0)

    lane = lax.iota(jnp.int32, _L)
    eo = lane >> 2
    fld = lane & 3

    def chunk(c, carry):
        base = wid * _EPW + c * _SCH
        pltpu.sync_copy(src_hbm.at[pl.ds(base, _SCH)], src_v)
        pltpu.sync_copy(pairs_hbm.at[pl.ds(base * 4, _SCH * 4)], pairs_v)

        def grp(j, icarry):
            vals = pairs_v[pl.ds(j * _L, _L)]
            s = plsc.load_gather(src_v, [j * 4 + eo])
            tgt = fld * _NP + s
            for e in range(4):
                plsc.addupdate_scatter(acc, [tgt], vals, mask=eo == e)
            return icarry

        lax.fori_loop(0, _SCH // 4, grp, 0)
        return carry

    lax.fori_loop(0, _SITER, chunk, 0)
    pltpu.sync_copy(acc, part_out.at[wid])


def _mlp_body(xs, xd, dist, vech, wrbf, brbf,
              e_Win, e_bin, e_Wh, e_bh, e_Wout, e_bout,
              f_Win, f_bin, f_Wh, f_bh, f_Wout, f_bout, out_ref):
    step = _RBF_R / (_NG - 1)
    offs = lax.broadcasted_iota(jnp.int32, (1, _NG), 1).astype(jnp.float32) * step
    coeff = -0.5 / step**2
    g = jnp.exp(coeff * (dist[...] - offs) ** 2)
    rbf = jnp.dot(g.astype(jnp.bfloat16), wrbf[...],
                  preferred_element_type=jnp.float32) + brbf[...]
    inp = jnp.concatenate(
        [xs[...], xd[...], rbf.astype(jnp.bfloat16)], axis=1)

    def res_mlp(Win, bin_, Wh, bh, Wout, bout):
        h = jnp.dot(inp, Win[...], preferred_element_type=jnp.float32) + bin_[...]
        h = jax.nn.silu(h)
        h2 = jnp.dot(h.astype(jnp.bfloat16), Wh[...],
                     preferred_element_type=jnp.float32) + bh[...]
        h = h + jax.nn.silu(h2)
        return jnp.dot(h.astype(jnp.bfloat16), Wout[...],
                       preferred_element_type=jnp.float32) + bout[...]

    ep = res_mlp(e_Win, e_bin, e_Wh, e_bh, e_Wout, e_bout) * (1.0 / (_AVG_LEN * _CONN))
    fp = res_mlp(f_Win, f_bin, f_Wh, f_bh, f_Wout, f_bout) * (1.0 / _CONN)
    mask0 = (lax.broadcasted_iota(jnp.int32, (1, 4), 1) == 0).astype(jnp.float32)
    out_ref[...] = ep * mask0 + fp * vech[...]


def _finish_body(x_ref, b_ref, energy_ref, ft_ref):
    xv = x_ref[...]
    e_node = jnp.sum(xv[:, 0:_NP], axis=0, keepdims=True)
    ft_ref[0:1, :] = jnp.sum(xv[:, _NP:2 * _NP], axis=0, keepdims=True)
    ft_ref[1:2, :] = jnp.sum(xv[:, 2 * _NP:3 * _NP], axis=0, keepdims=True)
    ft_ref[2:3, :] = jnp.sum(xv[:, 3 * _NP:4 * _NP], axis=0, keepdims=True)
    gid = lax.broadcasted_iota(jnp.int32, (_NGRAPH, 1), 0)
    onehot = (b_ref[...] == gid).astype(jnp.float32)
    energy_ref[...] = lax.dot_general(
        onehot, e_node, (((1,), (1,)), ((), ())),
        preferred_element_type=jnp.float32)


@functools.lru_cache(maxsize=None)
def _sc_calls():
    mesh = plsc.VectorSubcoreMesh(core_axis_name="c", subcore_axis_name="s")
    gather = pl.kernel(
        _gather_body,
        out_type=[jax.ShapeDtypeStruct((_E, _XW), jnp.int32),
                  jax.ShapeDtypeStruct((_E, _XW), jnp.int32)],
        mesh=mesh,
        compiler_params=pltpu.CompilerParams(use_tc_tiling_on_sc=False),
        scratch_types=[
            pltpu.VMEM((_GCH,), jnp.int32),
            pltpu.VMEM((_GCH,), jnp.int32),
            pltpu.VMEM((_GCH, _XW), jnp.int32),
            pltpu.VMEM((_GCH, _XW), jnp.int32),
            pltpu.SemaphoreType.DMA,
            pltpu.SemaphoreType.DMA,
        ],
    )
    scatter = pl.kernel(
        _scatter_body,
        out_type=[jax.ShapeDtypeStruct((_NW, _ACC), jnp.float32)],
        mesh=mesh,
        compiler_params=pltpu.CompilerParams(needs_layout_passes=False),
        scratch_types=[
            pltpu.VMEM((_ACC,), jnp.float32),
            pltpu.VMEM((_SCH * 4,), jnp.float32),
            pltpu.VMEM((_SCH,), jnp.int32),
        ],
    )
    return gather, scatter


def _mlp_call(xs, xd, dist2, vech4, *weights):
    grid = (_E // _BE,)
    edge_spec = lambda width: pl.BlockSpec((_BE, width), lambda i: (i, 0))
    w_spec = lambda a, b: pl.BlockSpec((a, b), lambda i: (0, 0))
    in_specs = [
        edge_spec(_EMBED), edge_spec(_EMBED), edge_spec(1), edge_spec(4),
        w_spec(_NG, _EMBED), w_spec(1, _EMBED),
        w_spec(3 * _EMBED, _HID), w_spec(1, _HID),
        w_spec(_HID, _HID), w_spec(1, _HID),
        w_spec(_HID, 1), w_spec(1, 1),
        w_spec(3 * _EMBED, _HID), w_spec(1, _HID),
        w_spec(_HID, _HID), w_spec(1, _HID),
        w_spec(_HID, 1), w_spec(1, 1),
    ]
    return pl.pallas_call(
        _mlp_body,
        grid=grid,
        in_specs=in_specs,
        out_specs=pl.BlockSpec((_BE, 4), lambda i: (i, 0)),
        out_shape=jax.ShapeDtypeStruct((_E, 4), jnp.float32),
    )(xs, xd, dist2, vech4, *weights)


def _finish_call(x, batch2):
    return pl.pallas_call(
        _finish_body,
        in_specs=[pl.BlockSpec((_NW, _ACC), lambda: (0, 0)),
                  pl.BlockSpec((1, _NP), lambda: (0, 0))],
        out_specs=[pl.BlockSpec((_NGRAPH, 1), lambda: (0, 0)),
                   pl.BlockSpec((3, _NP), lambda: (0, 0))],
        out_shape=[jax.ShapeDtypeStruct((_NGRAPH, 1), jnp.float32),
                   jax.ShapeDtypeStruct((3, _NP), jnp.float32)],
    )(x, batch2)


def kernel(x, edge_index, batch, dist, vec_hat,
           W_rbf, b_rbf,
           e_Win, e_bin, e_Wh, e_bh, e_Wout, e_bout,
           f_Win, f_bin, f_Wh, f_bh, f_Wout, f_bout):
    gather_call, scatter_call = _sc_calls()
    src = edge_index[0]
    dst = edge_index[1]
    x_i32 = lax.bitcast_convert_type(
        x.astype(jnp.bfloat16).reshape(_N, _XW, 2), jnp.int32)
    xs_i, xd_i = gather_call(x_i32, src, dst)
    xs = lax.bitcast_convert_type(xs_i, jnp.bfloat16).reshape(_E, _EMBED)
    xd = lax.bitcast_convert_type(xd_i, jnp.bfloat16).reshape(_E, _EMBED)
    dist2 = dist.reshape(_E, 1)
    vech4 = jnp.concatenate(
        [jnp.zeros((_E, 1), jnp.float32), vec_hat], axis=1)
    bf = jnp.bfloat16
    pairs = _mlp_call(
        xs, xd, dist2, vech4,
        W_rbf.astype(bf), b_rbf.reshape(1, _EMBED),
        e_Win.astype(bf), e_bin.reshape(1, _HID),
        e_Wh.astype(bf), e_bh.reshape(1, _HID),
        e_Wout.astype(bf), e_bout.reshape(1, 1),
        f_Win.astype(bf), f_bin.reshape(1, _HID),
        f_Wh.astype(bf), f_bh.reshape(1, _HID),
        f_Wout.astype(bf), f_bout.reshape(1, 1))
    (partials,) = scatter_call(pairs.reshape(_E * 4), src)
    batch_p = jnp.full((1, _NP), _NGRAPH, jnp.int32).at[0, :_N].set(batch)
    energy, ft = _finish_call(partials, batch_p)
    forces = ft[:, :_N].T
    return (energy, forces)


# speedup vs baseline: 2.0613x; 2.0613x over previous
import functools

import jax
import jax.numpy as jnp
from jax import lax
from jax.experimental import pallas as pl
from jax.experimental.pallas import tpu as pltpu
from jax.experimental.pallas import tpu_sc as plsc

_N = 10000
_E = 320000
_EMBED = 128
_HID = 256
_NG = 50
_RBF_R = 12.0
_AVG_LEN = 60.0
_CONN = 32.0
_NGRAPH = 64

_NC = 2
_NS = 16
_L = 16
_NW = _NC * _NS
_EPW = _E // _NW

_GCH = 80
_GITER = _EPW // _GCH

_SCH = 400
_SITER = _EPW // _SCH

_NP = 10240
_ACC = 4 * _NP

_BE = 512


_XW = _EMBED // 2


def _gather_body(x_hbm, src_hbm, dst_hbm, xs_out, xd_out,
                 idx_s, idx_d, rows_s, rows_d, sem_s, sem_d):
    wid = lax.axis_index("s") * _NC + lax.axis_index("c")

    def step(c, carry):
        base = wid * _EPW + c * _GCH
        pltpu.sync_copy(src_hbm.at[pl.ds(base, _GCH)], idx_s)
        pltpu.sync_copy(dst_hbm.at[pl.ds(base, _GCH)], idx_d)
        cp_s = pltpu.async_copy(x_hbm.at[idx_s], rows_s, sem_s)
        cp_d = pltpu.async_copy(x_hbm.at[idx_d], rows_d, sem_d)
        cp_s.wait()
        pltpu.sync_copy(rows_s, xs_out.at[pl.ds(base, _GCH)])
        cp_d.wait()
        pltpu.sync_copy(rows_d, xd_out.at[pl.ds(base, _GCH)])
        return carry

    lax.fori_loop(0, _GITER, step, 0)


def _scatter_body(pairs_hbm, src_hbm, part_out, acc, pairs_v, src_v):
    wid = lax.axis_index("s") * _NC + lax.axis_index("c")
    zero = jnp.zeros((_L,), jnp.float32)

    def zstep(i, carry):
        acc[pl.ds(i * _L, _L)] = zero
        return carry

    lax.fori_loop(0, _ACC // _L, zstep, 0)

    lane = lax.iota(jnp.int32, _L)
    eo = lane >> 2
    fld = lane & 3

    def chunk(c, carry):
        base = wid * _EPW + c * _SCH
        pltpu.sync_copy(src_hbm.at[pl.ds(base, _SCH)], src_v)
        pltpu.sync_copy(pairs_hbm.at[pl.ds(base * 4, _SCH * 4)], pairs_v)

        def grp(j, icarry):
            vals = pairs_v[pl.ds(j * _L, _L)]
            s = plsc.load_gather(src_v, [j * 4 + eo])
            tgt = fld * _NP + s
            for e in range(4):
                plsc.addupdate_scatter(acc, [tgt], vals, mask=eo == e)
            return icarry

        lax.fori_loop(0, _SCH // 4, grp, 0)
        return carry

    lax.fori_loop(0, _SITER, chunk, 0)
    pltpu.sync_copy(acc, part_out.at[wid])


def _mlp_body(xs, xd, dist, vech, wrbf, brbf,
              e_Win, e_bin, e_Wh, e_bh, e_Wout, e_bout,
              f_Win, f_bin, f_Wh, f_bh, f_Wout, f_bout, out_ref):
    step = _RBF_R / (_NG - 1)
    offs = lax.broadcasted_iota(jnp.int32, (1, _NG), 1).astype(jnp.float32) * step
    coeff = -0.5 / step**2
    g = jnp.exp(coeff * (dist[...] - offs) ** 2)
    rbf = jnp.dot(g.astype(jnp.bfloat16), wrbf[...],
                  preferred_element_type=jnp.float32) + brbf[...]
    inp = jnp.concatenate(
        [xs[...].astype(jnp.bfloat16), xd[...].astype(jnp.bfloat16),
         rbf.astype(jnp.bfloat16)], axis=1)

    def res_mlp(Win, bin_, Wh, bh, Wout, bout):
        h = jnp.dot(inp, Win[...], preferred_element_type=jnp.float32) + bin_[...]
        h = jax.nn.silu(h)
        h2 = jnp.dot(h.astype(jnp.bfloat16), Wh[...],
                     preferred_element_type=jnp.float32) + bh[...]
        h = h + jax.nn.silu(h2)
        return jnp.dot(h.astype(jnp.bfloat16), Wout[...],
                       preferred_element_type=jnp.float32) + bout[...]

    ep = res_mlp(e_Win, e_bin, e_Wh, e_bh, e_Wout, e_bout) * (1.0 / (_AVG_LEN * _CONN))
    fp = res_mlp(f_Win, f_bin, f_Wh, f_bh, f_Wout, f_bout) * (1.0 / _CONN)
    mask0 = (lax.broadcasted_iota(jnp.int32, (1, 4), 1) == 0).astype(jnp.float32)
    out_ref[...] = ep * mask0 + fp * vech[...]


def _finish_body(x_ref, b_ref, energy_ref, ft_ref):
    xv = x_ref[...]
    e_node = jnp.sum(xv[:, 0:_NP], axis=0, keepdims=True)
    ft_ref[0:1, :] = jnp.sum(xv[:, _NP:2 * _NP], axis=0, keepdims=True)
    ft_ref[1:2, :] = jnp.sum(xv[:, 2 * _NP:3 * _NP], axis=0, keepdims=True)
    ft_ref[2:3, :] = jnp.sum(xv[:, 3 * _NP:4 * _NP], axis=0, keepdims=True)
    gid = lax.broadcasted_iota(jnp.int32, (_NGRAPH, 1), 0)
    onehot = (b_ref[...] == gid).astype(jnp.float32)
    energy_ref[...] = lax.dot_general(
        onehot, e_node, (((1,), (1,)), ((), ())),
        preferred_element_type=jnp.float32)


@functools.lru_cache(maxsize=None)
def _sc_calls():
    mesh = plsc.VectorSubcoreMesh(core_axis_name="c", subcore_axis_name="s")
    gather = pl.kernel(
        _gather_body,
        out_type=[jax.ShapeDtypeStruct((_E, _EMBED), jnp.float32),
                  jax.ShapeDtypeStruct((_E, _EMBED), jnp.float32)],
        mesh=mesh,
        scratch_types=[
            pltpu.VMEM((_GCH,), jnp.int32),
            pltpu.VMEM((_GCH,), jnp.int32),
            pltpu.VMEM((_GCH, _EMBED), jnp.float32),
            pltpu.VMEM((_GCH, _EMBED), jnp.float32),
            pltpu.SemaphoreType.DMA,
            pltpu.SemaphoreType.DMA,
        ],
    )
    scatter = pl.kernel(
        _scatter_body,
        out_type=[jax.ShapeDtypeStruct((_NW, _ACC), jnp.float32)],
        mesh=mesh,
        compiler_params=pltpu.CompilerParams(needs_layout_passes=False),
        scratch_types=[
            pltpu.VMEM((_ACC,), jnp.float32),
            pltpu.VMEM((_SCH * 4,), jnp.float32),
            pltpu.VMEM((_SCH,), jnp.int32),
        ],
    )
    return gather, scatter


def _mlp_call(xs, xd, dist2, vech4, *weights):
    grid = (_E // _BE,)
    edge_spec = lambda width: pl.BlockSpec((_BE, width), lambda i: (i, 0))
    w_spec = lambda a, b: pl.BlockSpec((a, b), lambda i: (0, 0))
    in_specs = [
        edge_spec(_EMBED), edge_spec(_EMBED), edge_spec(1), edge_spec(4),
        w_spec(_NG, _EMBED), w_spec(1, _EMBED),
        w_spec(3 * _EMBED, _HID), w_spec(1, _HID),
        w_spec(_HID, _HID), w_spec(1, _HID),
        w_spec(_HID, 1), w_spec(1, 1),
        w_spec(3 * _EMBED, _HID), w_spec(1, _HID),
        w_spec(_HID, _HID), w_spec(1, _HID),
        w_spec(_HID, 1), w_spec(1, 1),
    ]
    return pl.pallas_call(
        _mlp_body,
        grid=grid,
        in_specs=in_specs,
        out_specs=pl.BlockSpec((_BE, 4), lambda i: (i, 0)),
        out_shape=jax.ShapeDtypeStruct((_E, 4), jnp.float32),
    )(xs, xd, dist2, vech4, *weights)


def _finish_call(x, batch2):
    return pl.pallas_call(
        _finish_body,
        in_specs=[pl.BlockSpec((_NW, _ACC), lambda: (0, 0)),
                  pl.BlockSpec((1, _NP), lambda: (0, 0))],
        out_specs=[pl.BlockSpec((_NGRAPH, 1), lambda: (0, 0)),
                   pl.BlockSpec((3, _NP), lambda: (0, 0))],
        out_shape=[jax.ShapeDtypeStruct((_NGRAPH, 1), jnp.float32),
                   jax.ShapeDtypeStruct((3, _NP), jnp.float32)],
    )(x, batch2)


def kernel(x, edge_index, batch, dist, vec_hat,
           W_rbf, b_rbf,
           e_Win, e_bin, e_Wh, e_bh, e_Wout, e_bout,
           f_Win, f_bin, f_Wh, f_bh, f_Wout, f_bout):
    gather_call, scatter_call = _sc_calls()
    src = edge_index[0]
    dst = edge_index[1]
    xs, xd = gather_call(x, src, dst)
    dist2 = dist.reshape(_E, 1)
    vech4 = jnp.concatenate(
        [jnp.zeros((_E, 1), jnp.float32), vec_hat], axis=1)
    bf = jnp.bfloat16
    pairs = _mlp_call(
        xs, xd, dist2, vech4,
        W_rbf.astype(bf), b_rbf.reshape(1, _EMBED),
        e_Win.astype(bf), e_bin.reshape(1, _HID),
        e_Wh.astype(bf), e_bh.reshape(1, _HID),
        e_Wout.astype(bf), e_bout.reshape(1, 1),
        f_Win.astype(bf), f_bin.reshape(1, _HID),
        f_Wh.astype(bf), f_bh.reshape(1, _HID),
        f_Wout.astype(bf), f_bout.reshape(1, 1))
    (partials,) = scatter_call(pairs.reshape(_E * 4), src)
    batch_p = jnp.full((1, _NP), _NGRAPH, jnp.int32).at[0, :_N].set(batch)
    energy, ft = _finish_call(partials, batch_p)
    forces = ft[:, :_N].T
    return (energy, forces)


# speedup vs baseline: 2.3576x; 1.1437x over previous
import functools

import jax
import jax.numpy as jnp
from jax import lax
from jax.experimental import pallas as pl
from jax.experimental.pallas import tpu as pltpu
from jax.experimental.pallas import tpu_sc as plsc

_N = 10000
_E = 320000
_EMBED = 128
_HID = 256
_NG = 50
_RBF_R = 12.0
_AVG_LEN = 60.0
_CONN = 32.0
_NGRAPH = 64

_NC = 2
_NS = 16
_L = 16
_NW = _NC * _NS
_EPW = _E // _NW

_GCH = 80
_GITER = _EPW // _GCH

_SCH = 400
_SITER = _EPW // _SCH

_NP = 10240
_ACC = 4 * _NP

_BE = 1280


_XW = _EMBED // 2


def _gather_body(x_hbm, src_hbm, dst_hbm, xs_out, xd_out,
                 idx_s, idx_d, rows_s, rows_d, sem_s, sem_d):
    wid = lax.axis_index("s") * _NC + lax.axis_index("c")

    def step(c, carry):
        base = wid * _EPW + c * _GCH
        pltpu.sync_copy(src_hbm.at[pl.ds(base, _GCH)], idx_s)
        pltpu.sync_copy(dst_hbm.at[pl.ds(base, _GCH)], idx_d)
        cp_s = pltpu.async_copy(x_hbm.at[idx_s], rows_s, sem_s)
        cp_d = pltpu.async_copy(x_hbm.at[idx_d], rows_d, sem_d)
        cp_s.wait()
        pltpu.sync_copy(rows_s, xs_out.at[pl.ds(base, _GCH)])
        cp_d.wait()
        pltpu.sync_copy(rows_d, xd_out.at[pl.ds(base, _GCH)])
        return carry

    lax.fori_loop(0, _GITER, step, 0)


def _scatter_body(pairs_hbm, src_hbm, part_out, acc, pairs_v, src_v):
    wid = lax.axis_index("s") * _NC + lax.axis_index("c")
    zero = jnp.zeros((_L,), jnp.float32)

    def zstep(i, carry):
        acc[pl.ds(i * _L, _L)] = zero
        return carry

    lax.fori_loop(0, _ACC // _L, zstep, 0)

    lane = lax.iota(jnp.int32, _L)
    eo = lane >> 2
    fld = lane & 3

    def chunk(c, carry):
        base = wid * _EPW + c * _SCH
        pltpu.sync_copy(src_hbm.at[pl.ds(base, _SCH)], src_v)
        pltpu.sync_copy(pairs_hbm.at[pl.ds(base * 4, _SCH * 4)], pairs_v)

        def grp(j, icarry):
            vals = pairs_v[pl.ds(j * _L, _L)]
            s = plsc.load_gather(src_v, [j * 4 + eo])
            tgt = fld * _NP + s
            for e in range(4):
                plsc.addupdate_scatter(acc, [tgt], vals, mask=eo == e)
            return icarry

        lax.fori_loop(0, _SCH // 4, grp, 0)
        return carry

    lax.fori_loop(0, _SITER, chunk, 0)
    pltpu.sync_copy(acc, part_out.at[wid])


def _mlp_body(xs, xd, dist, vech, wrbf, brbf,
              e_Win, e_bin, e_Wh, e_bh, e_Wout, e_bout,
              f_Win, f_bin, f_Wh, f_bh, f_Wout, f_bout, out_ref):
    step = _RBF_R / (_NG - 1)
    offs = lax.broadcasted_iota(jnp.int32, (1, _NG), 1).astype(jnp.float32) * step
    coeff = -0.5 / step**2
    g = jnp.exp(coeff * (dist[...] - offs) ** 2)
    rbf = jnp.dot(g.astype(jnp.bfloat16), wrbf[...],
                  preferred_element_type=jnp.float32) + brbf[...]
    inp = jnp.concatenate(
        [xs[...].astype(jnp.bfloat16), xd[...].astype(jnp.bfloat16),
         rbf.astype(jnp.bfloat16)], axis=1)

    def silu(v):
        return 0.5 * v * (1.0 + jnp.tanh(0.5 * v))

    def res_mlp(Win, bin_, Wh, bh, Wout, bout):
        h = jnp.dot(inp, Win[...], preferred_element_type=jnp.float32) + bin_[...]
        h = silu(h)
        h2 = jnp.dot(h.astype(jnp.bfloat16), Wh[...],
                     preferred_element_type=jnp.float32) + bh[...]
        h = h + silu(h2)
        return jnp.dot(h.astype(jnp.bfloat16), Wout[...],
                       preferred_element_type=jnp.float32) + bout[...]

    ep = res_mlp(e_Win, e_bin, e_Wh, e_bh, e_Wout, e_bout) * (1.0 / (_AVG_LEN * _CONN))
    fp = res_mlp(f_Win, f_bin, f_Wh, f_bh, f_Wout, f_bout) * (1.0 / _CONN)
    mask0 = (lax.broadcasted_iota(jnp.int32, (1, 4), 1) == 0).astype(jnp.float32)
    out_ref[...] = ep * mask0 + fp * vech[...]


def _finish_body(x_ref, b_ref, energy_ref, ft_ref):
    xv = x_ref[...]
    e_node = jnp.sum(xv[:, 0:_NP], axis=0, keepdims=True)
    ft_ref[0:1, :] = jnp.sum(xv[:, _NP:2 * _NP], axis=0, keepdims=True)
    ft_ref[1:2, :] = jnp.sum(xv[:, 2 * _NP:3 * _NP], axis=0, keepdims=True)
    ft_ref[2:3, :] = jnp.sum(xv[:, 3 * _NP:4 * _NP], axis=0, keepdims=True)
    gid = lax.broadcasted_iota(jnp.int32, (_NGRAPH, 1), 0)
    onehot = (b_ref[...] == gid).astype(jnp.float32)
    energy_ref[...] = lax.dot_general(
        onehot, e_node, (((1,), (1,)), ((), ())),
        preferred_element_type=jnp.float32)


@functools.lru_cache(maxsize=None)
def _sc_calls():
    mesh = plsc.VectorSubcoreMesh(core_axis_name="c", subcore_axis_name="s")
    gather = pl.kernel(
        _gather_body,
        out_type=[jax.ShapeDtypeStruct((_E, _EMBED), jnp.float32),
                  jax.ShapeDtypeStruct((_E, _EMBED), jnp.float32)],
        mesh=mesh,
        scratch_types=[
            pltpu.VMEM((_GCH,), jnp.int32),
            pltpu.VMEM((_GCH,), jnp.int32),
            pltpu.VMEM((_GCH, _EMBED), jnp.float32),
            pltpu.VMEM((_GCH, _EMBED), jnp.float32),
            pltpu.SemaphoreType.DMA,
            pltpu.SemaphoreType.DMA,
        ],
    )
    scatter = pl.kernel(
        _scatter_body,
        out_type=[jax.ShapeDtypeStruct((_NW, _ACC), jnp.float32)],
        mesh=mesh,
        compiler_params=pltpu.CompilerParams(needs_layout_passes=False),
        scratch_types=[
            pltpu.VMEM((_ACC,), jnp.float32),
            pltpu.VMEM((_SCH * 4,), jnp.float32),
            pltpu.VMEM((_SCH,), jnp.int32),
        ],
    )
    return gather, scatter


def _mlp_call(xs, xd, dist2, vech4, *weights):
    grid = (_E // _BE,)
    edge_spec = lambda width: pl.BlockSpec((_BE, width), lambda i: (i, 0))
    w_spec = lambda a, b: pl.BlockSpec((a, b), lambda i: (0, 0))
    in_specs = [
        edge_spec(_EMBED), edge_spec(_EMBED), edge_spec(1), edge_spec(4),
        w_spec(_NG, _EMBED), w_spec(1, _EMBED),
        w_spec(3 * _EMBED, _HID), w_spec(1, _HID),
        w_spec(_HID, _HID), w_spec(1, _HID),
        w_spec(_HID, 1), w_spec(1, 1),
        w_spec(3 * _EMBED, _HID), w_spec(1, _HID),
        w_spec(_HID, _HID), w_spec(1, _HID),
        w_spec(_HID, 1), w_spec(1, 1),
    ]
    return pl.pallas_call(
        _mlp_body,
        grid=grid,
        in_specs=in_specs,
        out_specs=pl.BlockSpec((_BE, 4), lambda i: (i, 0)),
        out_shape=jax.ShapeDtypeStruct((_E, 4), jnp.float32),
    )(xs, xd, dist2, vech4, *weights)


def _finish_call(x, batch2):
    return pl.pallas_call(
        _finish_body,
        in_specs=[pl.BlockSpec((_NW, _ACC), lambda: (0, 0)),
                  pl.BlockSpec((1, _NP), lambda: (0, 0))],
        out_specs=[pl.BlockSpec((_NGRAPH, 1), lambda: (0, 0)),
                   pl.BlockSpec((3, _NP), lambda: (0, 0))],
        out_shape=[jax.ShapeDtypeStruct((_NGRAPH, 1), jnp.float32),
                   jax.ShapeDtypeStruct((3, _NP), jnp.float32)],
    )(x, batch2)


def kernel(x, edge_index, batch, dist, vec_hat,
           W_rbf, b_rbf,
           e_Win, e_bin, e_Wh, e_bh, e_Wout, e_bout,
           f_Win, f_bin, f_Wh, f_bh, f_Wout, f_bout):
    gather_call, scatter_call = _sc_calls()
    src = edge_index[0]
    dst = edge_index[1]
    xs, xd = gather_call(x, src, dst)
    dist2 = dist.reshape(_E, 1)
    vech4 = jnp.concatenate(
        [jnp.zeros((_E, 1), jnp.float32), vec_hat], axis=1)
    bf = jnp.bfloat16
    pairs = _mlp_call(
        xs, xd, dist2, vech4,
        W_rbf.astype(bf), b_rbf.reshape(1, _EMBED),
        e_Win.astype(bf), e_bin.reshape(1, _HID),
        e_Wh.astype(bf), e_bh.reshape(1, _HID),
        e_Wout.astype(bf), e_bout.reshape(1, 1),
        f_Win.astype(bf), f_bin.reshape(1, _HID),
        f_Wh.astype(bf), f_bh.reshape(1, _HID),
        f_Wout.astype(bf), f_bout.reshape(1, 1))
    (partials,) = scatter_call(pairs.reshape(_E * 4), src)
    batch_p = jnp.full((1, _NP), _NGRAPH, jnp.int32).at[0, :_N].set(batch)
    energy, ft = _finish_call(partials, batch_p)
    forces = ft[:, :_N].T
    return (energy, forces)
